# Initial kernel scaffold; baseline (speedup 1.0000x reference)
import jax
import jax.numpy as jnp
from jax.experimental import pallas as pl


def kernel(output, target):
    raise NotImplementedError("write your pallas kernel here")


# speedup vs baseline: 28.8407x; 28.8407x over previous
import functools

import jax
import jax.numpy as jnp
import numpy as np
from jax.experimental import pallas as pl
from jax.experimental.pallas import tpu as pltpu

_NC = 20
_NA = 5
_ANCHORS = np.array(
    [1.3221, 1.73145, 3.19275, 4.00944, 5.05587, 8.09892, 9.47112, 4.84053,
     11.2364, 10.0071],
    dtype=np.float32).reshape(_NA, 2)
_THRESH = 0.6
_NH = 19
_NW = 19
_NPIX = _NH * _NW
_NT = 50


def _iou(b1x, b1y, b1w, b1h, b2x, b2y, b2w, b2h):
    mx = jnp.minimum(b1x - b1w / 2.0, b2x - b2w / 2.0)
    Mx = jnp.maximum(b1x + b1w / 2.0, b2x + b2w / 2.0)
    my = jnp.minimum(b1y - b1h / 2.0, b2y - b2h / 2.0)
    My = jnp.maximum(b1y + b1h / 2.0, b2y + b2h / 2.0)
    uw = Mx - mx
    uh = My - my
    cw = b1w + b2w - uw
    ch = b1h + b2h - uh
    carea = jnp.where((cw <= 0) | (ch <= 0), 0.0, cw * ch)
    uarea = b1w * b1h + b2w * b2h - carea
    return carea / uarea


def _const_vec(vals, shape, dim):
    idx = jax.lax.broadcasted_iota(jnp.int32, shape, dim)
    out = jnp.zeros(shape, jnp.float32)
    for i, v in enumerate(vals):
        out = jnp.where(idx == i, float(v), out)
    return out


def _region_loss_kernel(x_ref, tb_ref, tbt_ref, out_ref):
    x = x_ref[0]
    tb = tb_ref[0]
    tbt = tbt_ref[0]

    f32 = jnp.float32

    xs_c = tb[:, 1:2]
    ys_c = tb[:, 2:3]
    ws_c = tb[:, 3:4]
    hs_c = tb[:, 4:5]
    cls_c = tb[:, 0:1]
    gx_c = xs_c * _NW
    gy_c = ys_c * _NH
    gw_c = ws_c * _NW
    gh_c = hs_c * _NH
    gi_c = gx_c.astype(jnp.int32)
    gj_c = gy_c.astype(jnp.int32)

    xs_r = tbt[1:2, :]
    ws_r = tbt[3:4, :]
    hs_r = tbt[4:5, :]
    gx_r = xs_r * _NW
    gy_r = tbt[2:3, :] * _NH
    gw_r = ws_r * _NW
    gh_r = hs_r * _NH
    gi_r = gx_r.astype(jnp.int32)
    gj_r = gy_r.astype(jnp.int32)

    ti = jax.lax.broadcasted_iota(jnp.int32, (_NT, _NT), 0)
    tj = jax.lax.broadcasted_iota(jnp.int32, (_NT, _NT), 1)
    zeros_r = (xs_r == 0.0).astype(f32)
    zeros_c = (xs_c == 0.0).astype(f32)
    cnt_c = jnp.sum(jnp.where(tj <= ti, zeros_r, 0.0), axis=1, keepdims=True)
    valid_c = cnt_c == 0.0
    cnt_r = jnp.sum(jnp.where(ti <= tj, zeros_c, 0.0), axis=0, keepdims=True)
    valid_r = cnt_r == 0.0

    aw_r = _const_vec(_ANCHORS[:, 0], (1, _NA), 1)
    ah_r = _const_vec(_ANCHORS[:, 1], (1, _NA), 1)
    aw_c = _const_vec(_ANCHORS[:, 0], (_NA, 1), 0)
    ah_c = _const_vec(_ANCHORS[:, 1], (_NA, 1), 0)
    z = jnp.zeros((), f32)
    an_idx_r = jax.lax.broadcasted_iota(jnp.int32, (_NT, _NA), 1)
    tmp_c = _iou(z, z, aw_r, ah_r, z, z, gw_c, gh_c)
    m_c = jnp.max(tmp_c, axis=1, keepdims=True)
    bn_c = jnp.min(jnp.where(tmp_c == m_c, an_idx_r, _NA), axis=1,
                   keepdims=True)
    an_idx_c = jax.lax.broadcasted_iota(jnp.int32, (_NA, _NT), 0)
    tmp_r = _iou(z, z, aw_c, ah_c, z, z, gw_r, gh_r)
    m_r = jnp.max(tmp_r, axis=0, keepdims=True)
    bn_r = jnp.min(jnp.where(tmp_r == m_r, an_idx_c, _NA), axis=0,
                   keepdims=True)

    c_c = bn_c * _NPIX + gj_c * _NW + gi_c
    c_r = bn_r * _NPIX + gj_r * _NW + gi_r
    conflict = jnp.sum(
        jnp.where((tj > ti) & valid_r & (c_r == c_c), 1.0, 0.0),
        axis=1, keepdims=True)
    winner = valid_c & (conflict == 0.0)

    onehot_bn = (an_idx_r == bn_c).astype(f32)
    awn = jnp.sum(onehot_bn * aw_r, axis=1, keepdims=True)
    ahn = jnp.sum(onehot_bn * ah_r, axis=1, keepdims=True)

    tc0 = gx_c - gi_c.astype(f32)
    tc1 = gy_c - gj_c.astype(f32)
    tc2 = jnp.log(gw_c / awn)
    tc3 = jnp.log(gh_c / ahn)
    cm = 2.0 - ws_c * hs_c
    cls_idx = cls_c.astype(jnp.int32)
    cidx_r = jax.lax.broadcasted_iota(jnp.int32, (_NT, _NC), 1)
    onehot_cls = (cidx_r == cls_idx).astype(f32)

    col = jax.lax.broadcasted_iota(jnp.int32, (1, _NPIX), 1)
    grid_x = (col % _NW).astype(f32)
    grid_y = (col // _NW).astype(f32)
    p_r = jax.lax.broadcasted_iota(jnp.int32, (_NT, _NPIX), 1)

    conf_base = jnp.zeros((), f32)
    coord_base = jnp.zeros((), f32)
    conf_sel = jnp.zeros((_NT, 1), f32)
    ign_sel = jnp.zeros((_NT, 1), f32)
    bx_sel = jnp.zeros((_NT, 1), f32)
    by_sel = jnp.zeros((_NT, 1), f32)
    bw_sel = jnp.zeros((_NT, 1), f32)
    bh_sel = jnp.zeros((_NT, 1), f32)
    sx_sel = jnp.zeros((_NT, 1), f32)
    sy_sel = jnp.zeros((_NT, 1), f32)
    tw_sel = jnp.zeros((_NT, 1), f32)
    th_sel = jnp.zeros((_NT, 1), f32)
    lse_sel = jnp.zeros((_NT, 1), f32)
    clsv_sel = jnp.zeros((_NT, 1), f32)

    for a in range(_NA):
        base = a * (5 + _NC)
        tx = x[base + 0:base + 1, :]
        ty = x[base + 1:base + 2, :]
        tw = x[base + 2:base + 3, :]
        th = x[base + 3:base + 4, :]
        cf = x[base + 4:base + 5, :]
        cls = x[base + 5:base + 5 + _NC, :]

        sx = jax.nn.sigmoid(tx)
        sy = jax.nn.sigmoid(ty)
        conf = jax.nn.sigmoid(cf)
        bx = sx + grid_x
        by = sy + grid_y
        bw = jnp.exp(tw) * _ANCHORS[a, 0]
        bh = jnp.exp(th) * _ANCHORS[a, 1]

        coord_base += (jnp.sum((sx - 0.5) ** 2) + jnp.sum((sy - 0.5) ** 2)
                       + jnp.sum(tw * tw) + jnp.sum(th * th))

        ious = _iou(bx, by, bw, bh, gx_c, gy_c, gw_c, gh_c)
        cur_max = jnp.max(jnp.where(valid_c, ious, 0.0), axis=0,
                          keepdims=True)
        ign = (cur_max > _THRESH).astype(f32)
        conf_base += jnp.sum(jnp.where(cur_max > _THRESH, 0.0, conf * conf))

        eq = (c_c == a * _NPIX + p_r).astype(f32)
        conf_sel += jnp.sum(eq * conf, axis=1, keepdims=True)
        ign_sel += jnp.sum(eq * ign, axis=1, keepdims=True)
        bx_sel += jnp.sum(eq * bx, axis=1, keepdims=True)
        by_sel += jnp.sum(eq * by, axis=1, keepdims=True)
        bw_sel += jnp.sum(eq * bw, axis=1, keepdims=True)
        bh_sel += jnp.sum(eq * bh, axis=1, keepdims=True)
        sx_sel += jnp.sum(eq * sx, axis=1, keepdims=True)
        sy_sel += jnp.sum(eq * sy, axis=1, keepdims=True)
        tw_sel += jnp.sum(eq * tw, axis=1, keepdims=True)
        th_sel += jnp.sum(eq * th, axis=1, keepdims=True)

        cmax = jnp.max(cls, axis=0, keepdims=True)
        lse = jnp.log(jnp.sum(jnp.exp(cls - cmax), axis=0, keepdims=True)) + cmax
        lse_sel += jnp.sum(eq * lse, axis=1, keepdims=True)
        picked_tc = jax.lax.dot_general(
            eq, cls, (((1,), (1,)), ((), ())),
            preferred_element_type=f32)
        clsv_sel += jnp.sum(onehot_cls * picked_tc, axis=1, keepdims=True)

    iou_sel = _iou(gx_c, gy_c, gw_c, gh_c, bx_sel, by_sel, bw_sel, bh_sel)
    conf_corr = jnp.where(
        winner,
        25.0 * (conf_sel - iou_sel) ** 2 - (1.0 - ign_sel) * conf_sel ** 2,
        0.0)
    coord_corr = jnp.where(
        winner,
        cm * cm * ((sx_sel - tc0) ** 2 + (sy_sel - tc1) ** 2
                   + (tw_sel - tc2) ** 2 + (th_sel - tc3) ** 2)
        - 1e-4 * ((sx_sel - 0.5) ** 2 + (sy_sel - 0.5) ** 2
                  + tw_sel ** 2 + th_sel ** 2),
        0.0)
    cls_corr = jnp.where(winner, -(clsv_sel - lse_sel), 0.0)

    partial = (conf_base + 1e-4 * coord_base
               + jnp.sum(conf_corr) + jnp.sum(coord_corr) + jnp.sum(cls_corr))
    out_ref[0, 0, :] = jnp.full((128,), partial, f32)


@jax.jit
def kernel(output, target):
    nB = output.shape[0]
    x = output.reshape(nB, _NA * (5 + _NC), _NPIX)
    tb = target.reshape(nB, _NT, 5)
    tbt = jnp.transpose(tb, (0, 2, 1))
    partials = pl.pallas_call(
        _region_loss_kernel,
        grid=(nB,),
        in_specs=[
            pl.BlockSpec((1, _NA * (5 + _NC), _NPIX), lambda i: (i, 0, 0)),
            pl.BlockSpec((1, _NT, 5), lambda i: (i, 0, 0)),
            pl.BlockSpec((1, 5, _NT), lambda i: (i, 0, 0)),
        ],
        out_specs=pl.BlockSpec((1, 1, 128), lambda i: (i, 0, 0)),
        out_shape=jax.ShapeDtypeStruct((nB, 1, 128), jnp.float32),
        compiler_params=pltpu.CompilerParams(
            dimension_semantics=("arbitrary",)),
    )(x, tb, tbt)
    return jnp.sum(partials[:, 0, 0]) / nB
